# baseline (device time: 17536 ns/iter reference)
import jax
import jax.numpy as jnp
from jax import lax
from jax.experimental import pallas as pl
from jax.experimental.pallas import tpu as pltpu

M = 512
N_OUT = 512
MQ = 256
C = 16
R = MQ // C


def kernel(x):
    def body(x_ref, out_ref, recv_y_buf, recv_x_buf,
             y_send_sems, y_recv_sems, x_send_sems, x_recv_sems):
        my_x = lax.axis_index("x")
        my_y = lax.axis_index("y")
        other_x = 1 - my_x
        other_y = 1 - my_y

        barrier_sem = pltpu.get_barrier_semaphore()
        pl.semaphore_signal(barrier_sem, inc=1, device_id=(my_x, other_y),
                            device_id_type=pl.DeviceIdType.MESH)
        pl.semaphore_signal(barrier_sem, inc=1, device_id=(other_x, my_y),
                            device_id_type=pl.DeviceIdType.MESH)
        pl.semaphore_wait(barrier_sem, 2)

        y_rdmas = []
        for c in range(C):
            rdma = pltpu.make_async_remote_copy(
                src_ref=x_ref.at[0, pl.ds(my_x * MQ + c * R, R),
                                 pl.ds(other_y * N_OUT, N_OUT)],
                dst_ref=recv_y_buf.at[pl.ds(c * R, R)],
                send_sem=y_send_sems.at[c],
                recv_sem=y_recv_sems.at[c],
                device_id=(my_x, other_y),
                device_id_type=pl.DeviceIdType.MESH,
            )
            rdma.start()
            y_rdmas.append(rdma)

        LAG = 3

        def add_mine(c):
            out_ref[pl.ds(my_x * MQ + c * R, R), :] = (
                x_ref[0, pl.ds(my_x * MQ + c * R, R),
                      pl.ds(my_y * N_OUT, N_OUT)]
                + recv_y_buf[pl.ds(c * R, R), :]
            )

        def add_other(c):
            x_rdmas[c].wait_recv()
            out_ref[pl.ds(other_x * MQ + c * R, R), :] = (
                x_ref[0, pl.ds(other_x * MQ + c * R, R),
                      pl.ds(my_y * N_OUT, N_OUT)]
                + recv_x_buf[pl.ds(c * R, R), :]
            )

        x_rdmas = []
        for c in range(C):
            y_rdmas[c].wait_recv()
            rdma = pltpu.make_async_remote_copy(
                src_ref=recv_y_buf.at[pl.ds(c * R, R)],
                dst_ref=recv_x_buf.at[pl.ds(c * R, R)],
                send_sem=x_send_sems.at[c],
                recv_sem=x_recv_sems.at[c],
                device_id=(other_x, my_y),
                device_id_type=pl.DeviceIdType.MESH,
            )
            rdma.start()
            x_rdmas.append(rdma)
            add_mine(c)
            if c >= LAG:
                add_other(c - LAG)
        for c in range(C - LAG, C):
            add_other(c)

        for c in range(C):
            y_rdmas[c].wait_send()
            x_rdmas[c].wait_send()

    return pl.pallas_call(
        body,
        out_shape=jax.ShapeDtypeStruct((M, N_OUT), jnp.float32),
        in_specs=[pl.BlockSpec(memory_space=pltpu.VMEM)],
        out_specs=pl.BlockSpec(memory_space=pltpu.VMEM),
        scratch_shapes=[
            pltpu.VMEM((MQ, N_OUT), jnp.float32),
            pltpu.VMEM((MQ, N_OUT), jnp.float32),
            pltpu.SemaphoreType.DMA((C,)),
            pltpu.SemaphoreType.DMA((C,)),
            pltpu.SemaphoreType.DMA((C,)),
            pltpu.SemaphoreType.DMA((C,)),
        ],
        compiler_params=pltpu.CompilerParams(collective_id=0),
    )(x)


# device time: 15503 ns/iter; 1.1311x vs baseline; 1.1311x over previous
import jax
import jax.numpy as jnp
from jax import lax
from jax.experimental import pallas as pl
from jax.experimental.pallas import tpu as pltpu

M = 512
N_OUT = 512
MQ = 256
C = 16
R = MQ // C


def kernel(x):
    def body(x_ref, out_ref, send_buf, local_buf, acc, recv_y_buf, recv_x_buf,
             local_sems, out_sems,
             y_send_sems, y_recv_sems, x_send_sems, x_recv_sems):
        my_x = lax.axis_index("x")
        my_y = lax.axis_index("y")
        other_x = 1 - my_x
        other_y = 1 - my_y

        stage_send = pltpu.make_async_copy(
            x_ref.at[0, pl.ds(my_x * MQ, MQ), pl.ds(other_y * N_OUT, N_OUT)],
            send_buf,
            local_sems.at[0],
        )
        stage_send.start()
        stage_local = pltpu.make_async_copy(
            x_ref.at[0, :, pl.ds(my_y * N_OUT, N_OUT)],
            local_buf,
            local_sems.at[1],
        )
        stage_local.start()

        barrier_sem = pltpu.get_barrier_semaphore()
        pl.semaphore_signal(barrier_sem, inc=1, device_id=(my_x, other_y),
                            device_id_type=pl.DeviceIdType.MESH)
        pl.semaphore_signal(barrier_sem, inc=1, device_id=(other_x, my_y),
                            device_id_type=pl.DeviceIdType.MESH)
        pl.semaphore_wait(barrier_sem, 2)

        stage_send.wait()
        y_rdmas = []
        for c in range(C):
            rdma = pltpu.make_async_remote_copy(
                src_ref=send_buf.at[pl.ds(c * R, R)],
                dst_ref=recv_y_buf.at[pl.ds(c * R, R)],
                send_sem=y_send_sems.at[c],
                recv_sem=y_recv_sems.at[c],
                device_id=(my_x, other_y),
                device_id_type=pl.DeviceIdType.MESH,
            )
            rdma.start()
            y_rdmas.append(rdma)
        stage_local.wait()

        x_rdmas = []
        for c in range(C):
            y_rdmas[c].wait_recv()
            rdma = pltpu.make_async_remote_copy(
                src_ref=recv_y_buf.at[pl.ds(c * R, R)],
                dst_ref=recv_x_buf.at[pl.ds(c * R, R)],
                send_sem=x_send_sems.at[c],
                recv_sem=x_recv_sems.at[c],
                device_id=(other_x, my_y),
                device_id_type=pl.DeviceIdType.MESH,
            )
            rdma.start()
            x_rdmas.append(rdma)
            acc[pl.ds(my_x * MQ + c * R, R), :] = (
                local_buf[pl.ds(my_x * MQ + c * R, R), :]
                + recv_y_buf[pl.ds(c * R, R), :]
            )

        out_mine = pltpu.make_async_copy(
            acc.at[pl.ds(my_x * MQ, MQ)],
            out_ref.at[pl.ds(my_x * MQ, MQ)],
            out_sems.at[0],
        )
        out_mine.start()

        for c in range(C):
            x_rdmas[c].wait_recv()
            acc[pl.ds(other_x * MQ + c * R, R), :] = (
                local_buf[pl.ds(other_x * MQ + c * R, R), :]
                + recv_x_buf[pl.ds(c * R, R), :]
            )
        out_other = pltpu.make_async_copy(
            acc.at[pl.ds(other_x * MQ, MQ)],
            out_ref.at[pl.ds(other_x * MQ, MQ)],
            out_sems.at[1],
        )
        out_other.start()

        out_mine.wait()
        out_other.wait()
        for c in range(C):
            y_rdmas[c].wait_send()
            x_rdmas[c].wait_send()

    return pl.pallas_call(
        body,
        out_shape=jax.ShapeDtypeStruct((M, N_OUT), jnp.float32),
        in_specs=[pl.BlockSpec(memory_space=pltpu.MemorySpace.HBM)],
        out_specs=pl.BlockSpec(memory_space=pltpu.MemorySpace.HBM),
        scratch_shapes=[
            pltpu.VMEM((MQ, N_OUT), jnp.float32),
            pltpu.VMEM((M, N_OUT), jnp.float32),
            pltpu.VMEM((M, N_OUT), jnp.float32),
            pltpu.VMEM((MQ, N_OUT), jnp.float32),
            pltpu.VMEM((MQ, N_OUT), jnp.float32),
            pltpu.SemaphoreType.DMA((2,)),
            pltpu.SemaphoreType.DMA((2,)),
            pltpu.SemaphoreType.DMA((C,)),
            pltpu.SemaphoreType.DMA((C,)),
            pltpu.SemaphoreType.DMA((C,)),
            pltpu.SemaphoreType.DMA((C,)),
        ],
        compiler_params=pltpu.CompilerParams(collective_id=0),
    )(x)


# device time: 15207 ns/iter; 1.1532x vs baseline; 1.0195x over previous
import jax
import jax.numpy as jnp
from jax import lax
from jax.experimental import pallas as pl
from jax.experimental.pallas import tpu as pltpu

M = 512
N_OUT = 512
MQ = 256
C = 16
R = MQ // C


def kernel(x):
    def body(x_ref, out_ref, recv_y_buf, recv_x_buf,
             y_send_sems, y_recv_sems, x_send_sems, x_recv_sems):
        my_x = lax.axis_index("x")
        my_y = lax.axis_index("y")
        other_x = 1 - my_x
        other_y = 1 - my_y

        barrier_sem = pltpu.get_barrier_semaphore()
        pl.semaphore_signal(barrier_sem, inc=1, device_id=(my_x, other_y),
                            device_id_type=pl.DeviceIdType.MESH)
        pl.semaphore_signal(barrier_sem, inc=1, device_id=(other_x, my_y),
                            device_id_type=pl.DeviceIdType.MESH)
        pl.semaphore_wait(barrier_sem, 2)

        y_rdmas = []
        for c in range(C):
            rdma = pltpu.make_async_remote_copy(
                src_ref=x_ref.at[0, pl.ds(my_x * MQ + c * R, R),
                                 pl.ds(other_y * N_OUT, N_OUT)],
                dst_ref=recv_y_buf.at[pl.ds(c * R, R)],
                send_sem=y_send_sems.at[c],
                recv_sem=y_recv_sems.at[c],
                device_id=(my_x, other_y),
                device_id_type=pl.DeviceIdType.MESH,
            )
            rdma.start()
            y_rdmas.append(rdma)

        x_rdmas = []
        for c in range(C):
            y_rdmas[c].wait_recv()
            rdma = pltpu.make_async_remote_copy(
                src_ref=recv_y_buf.at[pl.ds(c * R, R)],
                dst_ref=recv_x_buf.at[pl.ds(c * R, R)],
                send_sem=x_send_sems.at[c],
                recv_sem=x_recv_sems.at[c],
                device_id=(other_x, my_y),
                device_id_type=pl.DeviceIdType.MESH,
            )
            rdma.start()
            x_rdmas.append(rdma)
            out_ref[pl.ds(my_x * MQ + c * R, R), :] = (
                x_ref[0, pl.ds(my_x * MQ + c * R, R),
                      pl.ds(my_y * N_OUT, N_OUT)]
                + recv_y_buf[pl.ds(c * R, R), :]
            )

        for c in range(C):
            x_rdmas[c].wait_recv()
            out_ref[pl.ds(other_x * MQ + c * R, R), :] = (
                x_ref[0, pl.ds(other_x * MQ + c * R, R),
                      pl.ds(my_y * N_OUT, N_OUT)]
                + recv_x_buf[pl.ds(c * R, R), :]
            )

        for c in range(C):
            y_rdmas[c].wait_send()
            x_rdmas[c].wait_send()

    return pl.pallas_call(
        body,
        out_shape=jax.ShapeDtypeStruct((M, N_OUT), jnp.float32),
        in_specs=[pl.BlockSpec(memory_space=pltpu.VMEM)],
        out_specs=pl.BlockSpec(memory_space=pltpu.VMEM),
        scratch_shapes=[
            pltpu.VMEM((MQ, N_OUT), jnp.float32),
            pltpu.VMEM((MQ, N_OUT), jnp.float32),
            pltpu.SemaphoreType.DMA((C,)),
            pltpu.SemaphoreType.DMA((C,)),
            pltpu.SemaphoreType.DMA((C,)),
            pltpu.SemaphoreType.DMA((C,)),
        ],
        compiler_params=pltpu.CompilerParams(collective_id=0),
    )(x)
